# Initial kernel scaffold; baseline (speedup 1.0000x reference)
import functools

import jax
import jax.numpy as jnp
from jax import lax
from jax.experimental import pallas as pl
from jax.experimental.pallas import tpu as pltpu
from jax.experimental.pallas import tpu_sc as plsc

N_NODES_C = 10000
N_EDGES_C = 320000
D_C = 128

_NC = 2
_NS = 16
_NW = _NC * _NS
_EPW = N_EDGES_C // _NW
_CH = 80
_NCHUNK = _EPW // _CH
_ROWS_PT = N_NODES_C // _NS


def _sc_agg_body(src_hbm, dst_hbm, h_hbm, zeros_hbm, out_hbm,
                 sidx_v, didx_v, rows_v, agg_sh, sem):
    c = lax.axis_index("c")
    s = lax.axis_index("s")
    w = s * _NC + c

    r0 = s * _ROWS_PT
    pltpu.sync_copy(zeros_hbm.at[pl.ds(r0, _ROWS_PT)],
                    agg_sh.at[pl.ds(r0, _ROWS_PT)])
    plsc.subcore_barrier()

    e_base = w * _EPW

    def body(i, carry):
        e0 = e_base + i * _CH
        pltpu.sync_copy(src_hbm.at[pl.ds(e0, _CH)], sidx_v)
        pltpu.sync_copy(dst_hbm.at[pl.ds(e0, _CH)], didx_v)
        pltpu.async_copy(h_hbm.at[sidx_v], rows_v, sem).wait()
        pltpu.sync_copy(rows_v, agg_sh.at[didx_v], add=True)
        return carry

    lax.fori_loop(0, _NCHUNK, body, 0)

    plsc.subcore_barrier()
    pltpu.sync_copy(agg_sh.at[pl.ds(r0, _ROWS_PT)],
                    out_hbm.at[c, pl.ds(r0, _ROWS_PT)])


@jax.jit
def _sc_agg(src, dst, h, zeros):
    mesh = plsc.VectorSubcoreMesh(core_axis_name="c", subcore_axis_name="s")
    k = pl.kernel(
        _sc_agg_body,
        out_type=jax.ShapeDtypeStruct((_NC, N_NODES_C, D_C), jnp.float32),
        mesh=mesh,
        scratch_types=[
            pltpu.VMEM((_CH,), jnp.int32),
            pltpu.VMEM((_CH,), jnp.int32),
            pltpu.VMEM((_CH, D_C), jnp.float32),
            pltpu.VMEM_SHARED((N_NODES_C, D_C), jnp.float32),
            pltpu.SemaphoreType.DMA,
        ],
    )
    return k(src, dst, h, zeros)


def _mlp_body(h_ref, a0_ref, a1_ref, w_ref, b_ref, eps_ref, out_ref):
    pre = (h_ref[...] * (1.0 + eps_ref[0, 0])
           + a0_ref[...] + a1_ref[...])
    y = jnp.dot(pre, w_ref[...], preferred_element_type=jnp.float32)
    out_ref[...] = jnp.maximum(y + b_ref[...], 0.0)


def _mlp_final_body(h_ref, a0_ref, a1_ref, w_ref, b_ref, eps_ref,
                    logits_ref, probs_ref):
    pre = (h_ref[...] * (1.0 + eps_ref[0, 0])
           + a0_ref[...] + a1_ref[...])
    y = jnp.dot(pre, w_ref[...], preferred_element_type=jnp.float32)
    logits = jnp.maximum(y + b_ref[...], 0.0)
    logits_ref[...] = logits
    m = jnp.max(logits, axis=-1, keepdims=True)
    e = jnp.exp(logits - m)
    probs_ref[...] = e / jnp.sum(e, axis=-1, keepdims=True)


_BN = 1000


def _row_spec():
    return pl.BlockSpec((_BN, D_C), lambda i: (i, 0))


def _full_spec(shape):
    return pl.BlockSpec(shape, lambda i: tuple(0 for _ in shape))


@jax.jit
def _tc_mlp(h, a0, a1, W, b, eps):
    return pl.pallas_call(
        _mlp_body,
        grid=(N_NODES_C // _BN,),
        in_specs=[_row_spec(), _row_spec(), _row_spec(),
                  _full_spec((D_C, D_C)), _full_spec((1, D_C)),
                  _full_spec((1, 1))],
        out_specs=_row_spec(),
        out_shape=jax.ShapeDtypeStruct((N_NODES_C, D_C), jnp.float32),
    )(h, a0, a1, W, b.reshape(1, D_C), eps.reshape(1, 1))


@jax.jit
def _tc_mlp_final(h, a0, a1, W, b, eps):
    return pl.pallas_call(
        _mlp_final_body,
        grid=(N_NODES_C // _BN,),
        in_specs=[_row_spec(), _row_spec(), _row_spec(),
                  _full_spec((D_C, D_C)), _full_spec((1, D_C)),
                  _full_spec((1, 1))],
        out_specs=(_row_spec(), _row_spec()),
        out_shape=(jax.ShapeDtypeStruct((N_NODES_C, D_C), jnp.float32),
                   jax.ShapeDtypeStruct((N_NODES_C, D_C), jnp.float32)),
    )(h, a0, a1, W, b.reshape(1, D_C), eps.reshape(1, 1))


def kernel(x, edge_index, W1, b1, eps1, W2, b2, eps2, W3, b3, eps3):
    src = edge_index[0].astype(jnp.int32)
    dst = edge_index[1].astype(jnp.int32)
    zeros = jnp.zeros((N_NODES_C, D_C), jnp.float32)

    agg = _sc_agg(src, dst, x, zeros)
    h = _tc_mlp(x, agg[0], agg[1], W1, b1, eps1)
    agg = _sc_agg(src, dst, h, zeros)
    h = _tc_mlp(h, agg[0], agg[1], W2, b2, eps2)
    agg = _sc_agg(src, dst, h, zeros)
    logits, probs = _tc_mlp_final(h, agg[0], agg[1], W3, b3, eps3)
    return (logits, probs)


# speedup vs baseline: 4.9204x; 4.9204x over previous
import functools

import jax
import jax.numpy as jnp
from jax import lax
from jax.experimental import pallas as pl
from jax.experimental.pallas import tpu as pltpu
from jax.experimental.pallas import tpu_sc as plsc

N_NODES_C = 10000
N_EDGES_C = 320000
D_C = 128

_NC = 2
_NS = 16
_NW = _NC * _NS
_EPW = N_EDGES_C // _NW
_CH = 80
_NCHUNK = _EPW // _CH
_ROWS_PT = 624
_ROWS_TAIL = N_NODES_C - _NS * _ROWS_PT


def _sc_agg_body(src_hbm, dst_hbm, h_hbm, zeros_hbm, out_hbm,
                 sidx_v, didx_v, rows_v, agg_sh, sem):
    c = lax.axis_index("c")
    s = lax.axis_index("s")
    w = s * _NC + c

    r0 = s * _ROWS_PT
    pltpu.sync_copy(zeros_hbm.at[pl.ds(r0, _ROWS_PT)],
                    agg_sh.at[pl.ds(r0, _ROWS_PT)])

    @pl.when(s == _NS - 1)
    def _():
        t0 = _NS * _ROWS_PT
        pltpu.sync_copy(zeros_hbm.at[pl.ds(t0, _ROWS_TAIL)],
                        agg_sh.at[pl.ds(t0, _ROWS_TAIL)])

    plsc.subcore_barrier()

    e_base = w * _EPW

    def body(i, carry):
        e0 = e_base + i * _CH
        pltpu.sync_copy(src_hbm.at[pl.ds(e0, _CH)], sidx_v)
        pltpu.sync_copy(dst_hbm.at[pl.ds(e0, _CH)], didx_v)
        pltpu.async_copy(h_hbm.at[sidx_v], rows_v, sem).wait()
        pltpu.sync_copy(rows_v, agg_sh.at[didx_v], add=True)
        return carry

    lax.fori_loop(0, _NCHUNK, body, 0)

    plsc.subcore_barrier()
    pltpu.sync_copy(agg_sh.at[pl.ds(r0, _ROWS_PT)],
                    out_hbm.at[c, pl.ds(r0, _ROWS_PT)])

    @pl.when(s == _NS - 1)
    def _():
        t0 = _NS * _ROWS_PT
        pltpu.sync_copy(agg_sh.at[pl.ds(t0, _ROWS_TAIL)],
                        out_hbm.at[c, pl.ds(t0, _ROWS_TAIL)])


@jax.jit
def _sc_agg(src, dst, h, zeros):
    mesh = plsc.VectorSubcoreMesh(core_axis_name="c", subcore_axis_name="s")
    k = pl.kernel(
        _sc_agg_body,
        out_type=jax.ShapeDtypeStruct((_NC, N_NODES_C, D_C), jnp.float32),
        mesh=mesh,
        scratch_types=[
            pltpu.VMEM((_CH,), jnp.int32),
            pltpu.VMEM((_CH,), jnp.int32),
            pltpu.VMEM((_CH, D_C), jnp.float32),
            pltpu.VMEM_SHARED((N_NODES_C, D_C), jnp.float32),
            pltpu.SemaphoreType.DMA,
        ],
    )
    return k(src, dst, h, zeros)


def _mlp_body(h_ref, a0_ref, a1_ref, w_ref, b_ref, eps_ref, out_ref):
    pre = (h_ref[...] * (1.0 + eps_ref[0, 0])
           + a0_ref[...] + a1_ref[...])
    y = jnp.dot(pre, w_ref[...], preferred_element_type=jnp.float32)
    out_ref[...] = jnp.maximum(y + b_ref[...], 0.0)


def _mlp_final_body(h_ref, a0_ref, a1_ref, w_ref, b_ref, eps_ref,
                    logits_ref, probs_ref):
    pre = (h_ref[...] * (1.0 + eps_ref[0, 0])
           + a0_ref[...] + a1_ref[...])
    y = jnp.dot(pre, w_ref[...], preferred_element_type=jnp.float32)
    logits = jnp.maximum(y + b_ref[...], 0.0)
    logits_ref[...] = logits
    m = jnp.max(logits, axis=-1, keepdims=True)
    e = jnp.exp(logits - m)
    probs_ref[...] = e / jnp.sum(e, axis=-1, keepdims=True)


_BN = 1000


def _row_spec():
    return pl.BlockSpec((_BN, D_C), lambda i: (i, 0))


def _full_spec(shape):
    return pl.BlockSpec(shape, lambda i: tuple(0 for _ in shape))


@jax.jit
def _tc_mlp(h, a0, a1, W, b, eps):
    return pl.pallas_call(
        _mlp_body,
        grid=(N_NODES_C // _BN,),
        in_specs=[_row_spec(), _row_spec(), _row_spec(),
                  _full_spec((D_C, D_C)), _full_spec((1, D_C)),
                  _full_spec((1, 1))],
        out_specs=_row_spec(),
        out_shape=jax.ShapeDtypeStruct((N_NODES_C, D_C), jnp.float32),
    )(h, a0, a1, W, b.reshape(1, D_C), eps.reshape(1, 1))


@jax.jit
def _tc_mlp_final(h, a0, a1, W, b, eps):
    return pl.pallas_call(
        _mlp_final_body,
        grid=(N_NODES_C // _BN,),
        in_specs=[_row_spec(), _row_spec(), _row_spec(),
                  _full_spec((D_C, D_C)), _full_spec((1, D_C)),
                  _full_spec((1, 1))],
        out_specs=(_row_spec(), _row_spec()),
        out_shape=(jax.ShapeDtypeStruct((N_NODES_C, D_C), jnp.float32),
                   jax.ShapeDtypeStruct((N_NODES_C, D_C), jnp.float32)),
    )(h, a0, a1, W, b.reshape(1, D_C), eps.reshape(1, 1))


def kernel(x, edge_index, W1, b1, eps1, W2, b2, eps2, W3, b3, eps3):
    src = edge_index[0].astype(jnp.int32)
    dst = edge_index[1].astype(jnp.int32)
    zeros = jnp.zeros((N_NODES_C, D_C), jnp.float32)

    agg = _sc_agg(src, dst, x, zeros)
    h = _tc_mlp(x, agg[0], agg[1], W1, b1, eps1)
    agg = _sc_agg(src, dst, h, zeros)
    h = _tc_mlp(h, agg[0], agg[1], W2, b2, eps2)
    agg = _sc_agg(src, dst, h, zeros)
    logits, probs = _tc_mlp_final(h, agg[0], agg[1], W3, b3, eps3)
    return (logits, probs)


# speedup vs baseline: 11.2507x; 2.2865x over previous
import functools

import jax
import jax.numpy as jnp
from jax import lax
from jax.experimental import pallas as pl
from jax.experimental.pallas import tpu as pltpu
from jax.experimental.pallas import tpu_sc as plsc

N_NODES_C = 10000
N_EDGES_C = 320000
D_C = 128

_NC = 2
_NS = 16
_NW = _NC * _NS
_EPW = N_EDGES_C // _NW
_CH = 80
_NCHUNK = _EPW // _CH
_ROWS_PT = 624
_ROWS_TAIL = N_NODES_C - _NS * _ROWS_PT


def _sc_agg_body(src_hbm, dst_hbm, h_hbm, zeros_hbm, out_hbm,
                 sidx_v, dbuf_a, dbuf_b, rows_a, rows_b, agg_sh,
                 sem_a, sem_b, semd_a, semd_b):
    c = lax.axis_index("c")
    s = lax.axis_index("s")
    w = s * _NC + c

    r0 = s * _ROWS_PT
    pltpu.sync_copy(zeros_hbm.at[pl.ds(r0, _ROWS_PT)],
                    agg_sh.at[pl.ds(r0, _ROWS_PT)])

    @pl.when(s == _NS - 1)
    def _():
        t0 = _NS * _ROWS_PT
        pltpu.sync_copy(zeros_hbm.at[pl.ds(t0, _ROWS_TAIL)],
                        agg_sh.at[pl.ds(t0, _ROWS_TAIL)])

    pltpu.sync_copy(src_hbm.at[pl.ds(w * _EPW, _EPW)], sidx_v)
    plsc.subcore_barrier()

    def gat(i, rows, sem):
        return pltpu.make_async_copy(
            h_hbm.at[sidx_v.at[pl.ds(i * _CH, _CH)]], rows, sem)

    def didx(i, dbuf, semd):
        return pltpu.make_async_copy(
            dst_hbm.at[pl.ds(w * _EPW + i * _CH, _CH)], dbuf, semd)

    def start(i, rows, sem, dbuf, semd):
        gat(i, rows, sem).start()
        didx(i, dbuf, semd).start()

    def finish(i, rows, sem, dbuf, semd):
        gat(i, rows, sem).wait()
        didx(i, dbuf, semd).wait()
        pltpu.sync_copy(rows, agg_sh.at[dbuf], add=True)

    a_args = (rows_a, sem_a, dbuf_a, semd_a)
    b_args = (rows_b, sem_b, dbuf_b, semd_b)

    start(0, *a_args)

    def outer(t, carry):
        i = 2 * t
        start(i + 1, *b_args)
        finish(i, *a_args)
        start(i + 2, *a_args)
        finish(i + 1, *b_args)
        return carry

    lax.fori_loop(0, (_NCHUNK - 1) // 2, outer, 0)
    finish(_NCHUNK - 1, *a_args)

    plsc.subcore_barrier()
    pltpu.sync_copy(agg_sh.at[pl.ds(r0, _ROWS_PT)],
                    out_hbm.at[c, pl.ds(r0, _ROWS_PT)])

    @pl.when(s == _NS - 1)
    def _():
        t0 = _NS * _ROWS_PT
        pltpu.sync_copy(agg_sh.at[pl.ds(t0, _ROWS_TAIL)],
                        out_hbm.at[c, pl.ds(t0, _ROWS_TAIL)])


@jax.jit
def _sc_agg(src, dst, h, zeros):
    mesh = plsc.VectorSubcoreMesh(core_axis_name="c", subcore_axis_name="s")
    k = pl.kernel(
        _sc_agg_body,
        out_type=jax.ShapeDtypeStruct((_NC, N_NODES_C, D_C), jnp.float32),
        mesh=mesh,
        scratch_types=[
            pltpu.VMEM((_EPW,), jnp.int32),
            pltpu.VMEM((_CH,), jnp.int32),
            pltpu.VMEM((_CH,), jnp.int32),
            pltpu.VMEM((_CH, D_C), jnp.float32),
            pltpu.VMEM((_CH, D_C), jnp.float32),
            pltpu.VMEM_SHARED((N_NODES_C, D_C), jnp.float32),
            pltpu.SemaphoreType.DMA,
            pltpu.SemaphoreType.DMA,
            pltpu.SemaphoreType.DMA,
            pltpu.SemaphoreType.DMA,
        ],
    )
    return k(src, dst, h, zeros)


def _mlp_body(h_ref, a0_ref, a1_ref, w_ref, b_ref, eps_ref, out_ref):
    pre = (h_ref[...] * (1.0 + eps_ref[0, 0])
           + a0_ref[...] + a1_ref[...])
    y = jnp.dot(pre, w_ref[...], preferred_element_type=jnp.float32)
    out_ref[...] = jnp.maximum(y + b_ref[...], 0.0)


def _mlp_final_body(h_ref, a0_ref, a1_ref, w_ref, b_ref, eps_ref,
                    logits_ref, probs_ref):
    pre = (h_ref[...] * (1.0 + eps_ref[0, 0])
           + a0_ref[...] + a1_ref[...])
    y = jnp.dot(pre, w_ref[...], preferred_element_type=jnp.float32)
    logits = jnp.maximum(y + b_ref[...], 0.0)
    logits_ref[...] = logits
    m = jnp.max(logits, axis=-1, keepdims=True)
    e = jnp.exp(logits - m)
    probs_ref[...] = e / jnp.sum(e, axis=-1, keepdims=True)


_BN = 1000


def _row_spec():
    return pl.BlockSpec((_BN, D_C), lambda i: (i, 0))


def _full_spec(shape):
    return pl.BlockSpec(shape, lambda i: tuple(0 for _ in shape))


@jax.jit
def _tc_mlp(h, a0, a1, W, b, eps):
    return pl.pallas_call(
        _mlp_body,
        grid=(N_NODES_C // _BN,),
        in_specs=[_row_spec(), _row_spec(), _row_spec(),
                  _full_spec((D_C, D_C)), _full_spec((1, D_C)),
                  _full_spec((1, 1))],
        out_specs=_row_spec(),
        out_shape=jax.ShapeDtypeStruct((N_NODES_C, D_C), jnp.float32),
    )(h, a0, a1, W, b.reshape(1, D_C), eps.reshape(1, 1))


@jax.jit
def _tc_mlp_final(h, a0, a1, W, b, eps):
    return pl.pallas_call(
        _mlp_final_body,
        grid=(N_NODES_C // _BN,),
        in_specs=[_row_spec(), _row_spec(), _row_spec(),
                  _full_spec((D_C, D_C)), _full_spec((1, D_C)),
                  _full_spec((1, 1))],
        out_specs=(_row_spec(), _row_spec()),
        out_shape=(jax.ShapeDtypeStruct((N_NODES_C, D_C), jnp.float32),
                   jax.ShapeDtypeStruct((N_NODES_C, D_C), jnp.float32)),
    )(h, a0, a1, W, b.reshape(1, D_C), eps.reshape(1, 1))


def kernel(x, edge_index, W1, b1, eps1, W2, b2, eps2, W3, b3, eps3):
    src = edge_index[0].astype(jnp.int32)
    dst = edge_index[1].astype(jnp.int32)
    zeros = jnp.zeros((N_NODES_C, D_C), jnp.float32)

    agg = _sc_agg(src, dst, x, zeros)
    h = _tc_mlp(x, agg[0], agg[1], W1, b1, eps1)
    agg = _sc_agg(src, dst, h, zeros)
    h = _tc_mlp(h, agg[0], agg[1], W2, b2, eps2)
    agg = _sc_agg(src, dst, h, zeros)
    logits, probs = _tc_mlp_final(h, agg[0], agg[1], W3, b3, eps3)
    return (logits, probs)
